# baseline (speedup 1.0000x reference)
import dataclasses
import functools

import jax
import jax.numpy as jnp
from jax.experimental import pallas as pl
from jax.experimental.pallas import tpu as pltpu
from jax.experimental.pallas import tpu_sc as plsc

N = 10000
D = 128
E = 320000
CHUNK = 125
NW = 32
EPW = E // NW
CPW = EPW // CHUNK
ROWS_PT = N // 16

def _sc_compiler_params():
    cp = pltpu.CompilerParams()
    if "needs_layout_passes" in pltpu.CompilerParams.__dataclass_fields__:
        cp = dataclasses.replace(cp, needs_layout_passes=False)
    return cp


@functools.cache
def _mesh():
    return plsc.VectorSubcoreMesh(core_axis_name="c", subcore_axis_name="s")


@functools.cache
def _make_sc_degree():
  @functools.partial(
      pl.kernel,
      out_type=jax.ShapeDtypeStruct((2, N), jnp.float32),
      mesh=_mesh(),
      scratch_types=[
          pltpu.VMEM((CPW, CHUNK), jnp.int32),
          pltpu.VMEM((128,), jnp.float32),
          pltpu.VMEM_SHARED((N,), jnp.float32),
          pltpu.SemaphoreType.DMA,
      ],
  )
  def _sc_degree(dst_hbm, ones_hbm, zero_hbm, out_hbm, idx_v, ones_v, acc_sp, sem):
      core = jax.lax.axis_index("c")
      tile = jax.lax.axis_index("s")
      w = core * 16 + tile
      pltpu.sync_copy(dst_hbm.at[pl.ds(w * CPW, CPW)], idx_v)
      pltpu.sync_copy(ones_hbm, ones_v)

      @pl.when(tile == 0)
      def _():
          pltpu.sync_copy(zero_hbm, acc_sp)

      plsc.subcore_barrier()

      @pl.loop(0, CPW, step=8)
      def _(j):
          for k in range(8):
              pltpu.async_copy(ones_v.at[pl.ds(0, CHUNK)],
                               acc_sp.at[idx_v.at[j + k]], sem, add=True)
          for k in range(8):
              pltpu.make_async_copy(ones_v.at[pl.ds(0, CHUNK)],
                                    acc_sp.at[idx_v.at[j + k]], sem).wait()

      plsc.subcore_barrier()

      @pl.when(tile == 0)
      def _():
          pltpu.sync_copy(acc_sp, out_hbm.at[core])


  return _sc_degree


@functools.cache
def _make_sc_agg(d):
    @functools.partial(
        pl.kernel,
        out_type=jax.ShapeDtypeStruct((2, N, d), jnp.float32),
        mesh=_mesh(),
        scratch_types=[
            pltpu.VMEM((CPW, CHUNK), jnp.int32),
            pltpu.VMEM((40, CHUNK), jnp.int32),
            pltpu.VMEM((CHUNK, d), jnp.float32),
            pltpu.VMEM((CHUNK, d), jnp.float32),
            pltpu.VMEM_SHARED((N, d), jnp.float32),
            pltpu.SemaphoreType.DMA,
            pltpu.SemaphoreType.DMA,
            pltpu.SemaphoreType.DMA,
            pltpu.SemaphoreType.DMA,
        ],
    )
    def _agg(g_hbm, src_hbm, dst_hbm, zero_hbm, out_hbm,
             src_v, dst_v, rows0, rows1, acc_sp, sem0, sem1, sem2, sem3):
        core = jax.lax.axis_index("c")
        tile = jax.lax.axis_index("s")
        w = core * 16 + tile
        pltpu.sync_copy(src_hbm.at[pl.ds(w * CPW, CPW)], src_v)
        base = pl.multiple_of(tile * 632, 8)

        @pl.when(tile < 15)
        def _():
            pltpu.sync_copy(zero_hbm.at[pl.ds(base, 632)],
                            acc_sp.at[pl.ds(base, 632)])

        @pl.when(tile == 15)
        def _():
            pltpu.sync_copy(zero_hbm.at[pl.ds(15 * 632, N - 15 * 632)],
                            acc_sp.at[pl.ds(15 * 632, N - 15 * 632)])

        pltpu.async_copy(g_hbm.at[src_v.at[0]], rows0, sem0)
        pltpu.async_copy(g_hbm.at[src_v.at[1]], rows1, sem1)
        plsc.subcore_barrier()

        @pl.loop(0, CPW, step=2)
        def _(j):
            q = j // 40

            @pl.when(j == q * 40)
            def _():
                off = pl.multiple_of(w * CPW + q * 40, 8)
                pltpu.sync_copy(dst_hbm.at[pl.ds(off, 40)], dst_v)

            jd = j - q * 40
            pltpu.make_async_copy(g_hbm.at[src_v.at[j]], rows0, sem0).wait()
            pltpu.async_copy(rows0, acc_sp.at[dst_v.at[jd]], sem2, add=True)
            pltpu.make_async_copy(g_hbm.at[src_v.at[j + 1]], rows1, sem1).wait()
            pltpu.async_copy(rows1, acc_sp.at[dst_v.at[jd + 1]], sem3, add=True)
            pltpu.make_async_copy(rows0, acc_sp.at[dst_v.at[jd]], sem2).wait()

            @pl.when(j + 2 < CPW)
            def _():
                pltpu.async_copy(g_hbm.at[src_v.at[j + 2]], rows0, sem0)

            pltpu.make_async_copy(rows1, acc_sp.at[dst_v.at[jd + 1]], sem3).wait()

            @pl.when(j + 3 < CPW)
            def _():
                pltpu.async_copy(g_hbm.at[src_v.at[j + 3]], rows1, sem1)

        plsc.subcore_barrier()

        @pl.when(tile < 15)
        def _():
            pltpu.sync_copy(acc_sp.at[pl.ds(base, 632)],
                            out_hbm.at[core, pl.ds(base, 632)])

        @pl.when(tile == 15)
        def _():
            pltpu.sync_copy(acc_sp.at[pl.ds(15 * 632, N - 15 * 632)],
                            out_hbm.at[core, pl.ds(15 * 632, N - 15 * 632)])

    return _agg


@functools.cache
def _make_sc_agg1d():
    @functools.partial(
        pl.kernel,
        out_type=jax.ShapeDtypeStruct((2, N + 8), jnp.float32),
        mesh=_mesh(),
        scratch_types=[
            pltpu.VMEM((80, 128), jnp.int32),
            pltpu.VMEM((80, 128), jnp.int32),
            pltpu.VMEM((N,), jnp.float32),
            pltpu.VMEM((80, 128), jnp.float32),
            pltpu.VMEM_SHARED((N + 8,), jnp.float32),
            pltpu.SemaphoreType.DMA,
        ],
        compiler_params=_sc_compiler_params(),
    )
    def _agg1d(y_hbm, src_hbm, dst_hbm, zero_hbm, out_hbm,
               src_v, dst_v, y_l, vals_v, acc_sp, sem):
        core = jax.lax.axis_index("c")
        tile = jax.lax.axis_index("s")
        w = core * 16 + tile
        pltpu.sync_copy(src_hbm.at[pl.ds(w * 80, 80)], src_v)
        pltpu.sync_copy(dst_hbm.at[pl.ds(w * 80, 80)], dst_v)
        pltpu.sync_copy(y_hbm, y_l)

        @pl.when(tile == 0)
        def _():
            pltpu.sync_copy(zero_hbm, acc_sp)

        @pl.loop(0, 80)
        def _(j):
            for k in range(8):
                idx = src_v[j, pl.ds(k * 16, 16)]
                vals_v[j, pl.ds(k * 16, 16)] = plsc.load_gather(y_l, [idx])

        plsc.subcore_barrier()

        @pl.loop(0, 80, step=8)
        def _(j):
            for k in range(8):
                pltpu.async_copy(vals_v.at[j + k], acc_sp.at[dst_v.at[j + k]],
                                 sem, add=True)
            for k in range(8):
                pltpu.make_async_copy(vals_v.at[j + k],
                                      acc_sp.at[dst_v.at[j + k]], sem).wait()

        plsc.subcore_barrier()

        @pl.when(tile == 0)
        def _():
            pltpu.sync_copy(acc_sp, out_hbm.at[core])

    return _agg1d


def _tc_matmul(x, W1):
    def body(x_ref, w_ref, o_ref):
        o_ref[...] = jnp.dot(x_ref[...], w_ref[...],
                             preferred_element_type=jnp.float32)

    return pl.pallas_call(
        body, out_shape=jax.ShapeDtypeStruct((N, D), jnp.float32))(x, W1)


def _tc_scale(h, degp):
    def body(h_ref, degp_ref, o_ref):
        dinv = jax.lax.rsqrt(degp_ref[0] + degp_ref[1] + 1.0)
        o_ref[...] = h_ref[...] * dinv

    return pl.pallas_call(
        body, out_shape=jax.ShapeDtypeStruct((N, D), jnp.float32))(h, degp)


def _tc_layer(p, g1, degp, b1r, w2r):
    def body(p_ref, g_ref, degp_ref, b1_ref, w2_ref, o_ref):
        dinv = jax.lax.rsqrt(degp_ref[0] + degp_ref[1] + 1.0)
        acc = p_ref[0] + p_ref[1] + g_ref[...]
        h1 = jnp.maximum(acc * dinv + b1_ref[...], 0.0)
        o_ref[...] = jnp.sum(h1 * w2_ref[...], axis=1, keepdims=True) * dinv

    return pl.pallas_call(
        body, out_shape=jax.ShapeDtypeStruct((N, 1), jnp.float32))(
            p, g1, degp, b1r, w2r)


def _tc_final(q, y1, degp, b2r):
    def body(q_ref, y_ref, degp_ref, b2_ref, o_ref):
        dinv = jax.lax.rsqrt(degp_ref[0] + degp_ref[1] + 1.0)
        tot = q_ref[0] + q_ref[1] + y_ref[...]
        o_ref[...] = jax.nn.sigmoid(tot * dinv + b2_ref[...])

    return pl.pallas_call(
        body, out_shape=jax.ShapeDtypeStruct((N, 1), jnp.float32))(
            q, y1, degp, b2r)


def kernel(x, edge_index, W1, b1, W2, b2):
    ei = edge_index.astype(jnp.int32)
    src = ei[0].reshape(NW * CPW, CHUNK)
    dst = ei[1].reshape(NW * CPW, CHUNK)
    npad = 2560 * 128 - E
    rng = jnp.arange(npad, dtype=jnp.int32)
    src2 = jnp.concatenate([ei[0], rng % 16]).reshape(2560, 128)
    dst2 = jnp.concatenate([ei[1], N + (rng % 8)]).reshape(2560, 128)
    ones_n = jnp.ones((128,), jnp.float32)
    zero_n = jnp.zeros((N,), jnp.float32)
    zero_n8 = jnp.zeros((N + 8,), jnp.float32)
    zero_nd = jnp.zeros((N, D), jnp.float32)

    degp = _make_sc_degree()(dst, ones_n, zero_n)
    h = _tc_matmul(x, W1)
    degp3 = degp.reshape(2, N, 1)
    g1 = _tc_scale(h, degp3)
    p = _make_sc_agg(D)(g1, src, dst, zero_nd)
    y1 = _tc_layer(p, g1, degp3, b1.reshape(1, D), W2.reshape(1, D))
    q = _make_sc_agg1d()(y1.reshape(N), src2, dst2, zero_n8)
    q = q[:, :N].reshape(2, N, 1)
    return _tc_final(q, y1, degp3, b2.reshape(1, 1))


# speedup vs baseline: 1.1624x; 1.1624x over previous
import dataclasses
import functools

import jax
import jax.numpy as jnp
from jax.experimental import pallas as pl
from jax.experimental.pallas import tpu as pltpu
from jax.experimental.pallas import tpu_sc as plsc

N = 10000
D = 128
E = 320000
CHUNK = 125
NW = 32
EPW = E // NW
CPW = EPW // CHUNK
ROWS_PT = N // 16

def _sc_compiler_params():
    cp = pltpu.CompilerParams()
    if "needs_layout_passes" in pltpu.CompilerParams.__dataclass_fields__:
        cp = dataclasses.replace(cp, needs_layout_passes=False)
    return cp


@functools.cache
def _mesh():
    return plsc.VectorSubcoreMesh(core_axis_name="c", subcore_axis_name="s")


@functools.cache
def _make_sc_degree():
  @functools.partial(
      pl.kernel,
      out_type=jax.ShapeDtypeStruct((2, N), jnp.float32),
      mesh=_mesh(),
      scratch_types=[
          pltpu.VMEM((CPW, CHUNK), jnp.int32),
          pltpu.VMEM((128,), jnp.float32),
          pltpu.VMEM_SHARED((N,), jnp.float32),
          pltpu.SemaphoreType.DMA,
      ],
  )
  def _sc_degree(dst_hbm, ones_hbm, zero_hbm, out_hbm, idx_v, ones_v, acc_sp, sem):
      core = jax.lax.axis_index("c")
      tile = jax.lax.axis_index("s")
      w = core * 16 + tile
      pltpu.sync_copy(dst_hbm.at[pl.ds(w * CPW, CPW)], idx_v)
      pltpu.sync_copy(ones_hbm, ones_v)

      @pl.when(tile == 0)
      def _():
          pltpu.sync_copy(zero_hbm, acc_sp)

      plsc.subcore_barrier()

      @pl.loop(0, CPW, step=8)
      def _(j):
          for k in range(8):
              pltpu.async_copy(ones_v.at[pl.ds(0, CHUNK)],
                               acc_sp.at[idx_v.at[j + k]], sem, add=True)
          for k in range(8):
              pltpu.make_async_copy(ones_v.at[pl.ds(0, CHUNK)],
                                    acc_sp.at[idx_v.at[j + k]], sem).wait()

      plsc.subcore_barrier()

      @pl.when(tile == 0)
      def _():
          pltpu.sync_copy(acc_sp, out_hbm.at[core])


  return _sc_degree


@functools.cache
def _make_sc_agg(d):
    @functools.partial(
        pl.kernel,
        out_type=jax.ShapeDtypeStruct((2, N, d), jnp.float32),
        mesh=_mesh(),
        scratch_types=[
            pltpu.VMEM((CPW, CHUNK), jnp.int32),
            pltpu.VMEM((40, CHUNK), jnp.int32),
            pltpu.VMEM((CHUNK, d), jnp.float32),
            pltpu.VMEM((CHUNK, d), jnp.float32),
            pltpu.VMEM_SHARED((N, d), jnp.float32),
            pltpu.SemaphoreType.DMA,
            pltpu.SemaphoreType.DMA,
        ],
    )
    def _agg(g_hbm, src_hbm, dst_hbm, zero_hbm, out_hbm,
             src_v, dst_v, rows0, rows1, acc_sp, sem0, sem1):
        core = jax.lax.axis_index("c")
        tile = jax.lax.axis_index("s")
        w = core * 16 + tile
        pltpu.sync_copy(src_hbm.at[pl.ds(w * CPW, CPW)], src_v)
        base = pl.multiple_of(tile * 632, 8)

        @pl.when(tile < 15)
        def _():
            pltpu.sync_copy(zero_hbm.at[pl.ds(base, 632)],
                            acc_sp.at[pl.ds(base, 632)])

        @pl.when(tile == 15)
        def _():
            pltpu.sync_copy(zero_hbm.at[pl.ds(15 * 632, N - 15 * 632)],
                            acc_sp.at[pl.ds(15 * 632, N - 15 * 632)])

        pltpu.async_copy(g_hbm.at[src_v.at[0]], rows0, sem0)
        plsc.subcore_barrier()

        @pl.loop(0, CPW, step=2)
        def _(j):
            q = j // 40

            @pl.when(j == q * 40)
            def _():
                off = pl.multiple_of(w * CPW + q * 40, 8)
                pltpu.sync_copy(dst_hbm.at[pl.ds(off, 40)], dst_v)

            jd = j - q * 40
            pltpu.async_copy(g_hbm.at[src_v.at[j + 1]], rows1, sem1)
            pltpu.make_async_copy(g_hbm.at[src_v.at[j]], rows0, sem0).wait()
            pltpu.sync_copy(rows0, acc_sp.at[dst_v.at[jd]], add=True)

            @pl.when(j + 2 < CPW)
            def _():
                pltpu.async_copy(g_hbm.at[src_v.at[j + 2]], rows0, sem0)

            pltpu.make_async_copy(g_hbm.at[src_v.at[j + 1]], rows1, sem1).wait()
            pltpu.sync_copy(rows1, acc_sp.at[dst_v.at[jd + 1]], add=True)

        plsc.subcore_barrier()

        @pl.when(tile < 15)
        def _():
            pltpu.sync_copy(acc_sp.at[pl.ds(base, 632)],
                            out_hbm.at[core, pl.ds(base, 632)])

        @pl.when(tile == 15)
        def _():
            pltpu.sync_copy(acc_sp.at[pl.ds(15 * 632, N - 15 * 632)],
                            out_hbm.at[core, pl.ds(15 * 632, N - 15 * 632)])

    return _agg


@functools.cache
def _make_sc_agg1d():
    @functools.partial(
        pl.kernel,
        out_type=jax.ShapeDtypeStruct((2, N + 8), jnp.float32),
        mesh=_mesh(),
        scratch_types=[
            pltpu.VMEM((80, 128), jnp.int32),
            pltpu.VMEM((80, 128), jnp.int32),
            pltpu.VMEM((N,), jnp.float32),
            pltpu.VMEM((80, 128), jnp.float32),
            pltpu.VMEM_SHARED((N + 8,), jnp.float32),
            pltpu.SemaphoreType.DMA,
        ],
        compiler_params=_sc_compiler_params(),
    )
    def _agg1d(y_hbm, src_hbm, dst_hbm, zero_hbm, out_hbm,
               src_v, dst_v, y_l, vals_v, acc_sp, sem):
        core = jax.lax.axis_index("c")
        tile = jax.lax.axis_index("s")
        w = core * 16 + tile
        pltpu.sync_copy(src_hbm.at[pl.ds(w * 80, 80)], src_v)
        pltpu.sync_copy(dst_hbm.at[pl.ds(w * 80, 80)], dst_v)
        pltpu.sync_copy(y_hbm, y_l)

        @pl.when(tile == 0)
        def _():
            pltpu.sync_copy(zero_hbm, acc_sp)

        @pl.loop(0, 80)
        def _(j):
            for k in range(8):
                idx = src_v[j, pl.ds(k * 16, 16)]
                vals_v[j, pl.ds(k * 16, 16)] = plsc.load_gather(y_l, [idx])

        plsc.subcore_barrier()

        @pl.loop(0, 80, step=8)
        def _(j):
            for k in range(8):
                pltpu.async_copy(vals_v.at[j + k], acc_sp.at[dst_v.at[j + k]],
                                 sem, add=True)
            for k in range(8):
                pltpu.make_async_copy(vals_v.at[j + k],
                                      acc_sp.at[dst_v.at[j + k]], sem).wait()

        plsc.subcore_barrier()

        @pl.when(tile == 0)
        def _():
            pltpu.sync_copy(acc_sp, out_hbm.at[core])

    return _agg1d


def _tc_matmul_scale(x, W1, degp):
    def body(x_ref, w_ref, degp_ref, o_ref):
        dinv = jax.lax.rsqrt(degp_ref[0] + degp_ref[1] + 1.0)
        h = jnp.dot(x_ref[...], w_ref[...], preferred_element_type=jnp.float32)
        o_ref[...] = h * dinv

    return pl.pallas_call(
        body, out_shape=jax.ShapeDtypeStruct((N, D), jnp.float32))(x, W1, degp)


def _tc_layer(p, g1, degp, b1r, w2r):
    def body(p_ref, g_ref, degp_ref, b1_ref, w2_ref, o_ref):
        dinv = jax.lax.rsqrt(degp_ref[0] + degp_ref[1] + 1.0)
        acc = p_ref[0] + p_ref[1] + g_ref[...]
        h1 = jnp.maximum(acc * dinv + b1_ref[...], 0.0)
        o_ref[...] = jnp.sum(h1 * w2_ref[...], axis=1, keepdims=True) * dinv

    return pl.pallas_call(
        body, out_shape=jax.ShapeDtypeStruct((N, 1), jnp.float32))(
            p, g1, degp, b1r, w2r)


def _tc_final(q, y1, degp, b2r):
    def body(q_ref, y_ref, degp_ref, b2_ref, o_ref):
        dinv = jax.lax.rsqrt(degp_ref[0] + degp_ref[1] + 1.0)
        tot = q_ref[0] + q_ref[1] + y_ref[...]
        o_ref[...] = jax.nn.sigmoid(tot * dinv + b2_ref[...])

    return pl.pallas_call(
        body, out_shape=jax.ShapeDtypeStruct((N, 1), jnp.float32))(
            q, y1, degp, b2r)


def kernel(x, edge_index, W1, b1, W2, b2):
    ei = edge_index.astype(jnp.int32)
    src = ei[0].reshape(NW * CPW, CHUNK)
    dst = ei[1].reshape(NW * CPW, CHUNK)
    npad = 2560 * 128 - E
    rng = jnp.arange(npad, dtype=jnp.int32)
    src2 = jnp.concatenate([ei[0], rng % 16]).reshape(2560, 128)
    dst2 = jnp.concatenate([ei[1], N + (rng % 8)]).reshape(2560, 128)
    ones_n = jnp.ones((128,), jnp.float32)
    zero_n = jnp.zeros((N,), jnp.float32)
    zero_n8 = jnp.zeros((N + 8,), jnp.float32)
    zero_nd = jnp.zeros((N, D), jnp.float32)

    degp = _make_sc_degree()(dst, ones_n, zero_n)
    degp3 = degp.reshape(2, N, 1)
    g1 = _tc_matmul_scale(x, W1, degp3)
    p = _make_sc_agg(D)(g1, src, dst, zero_nd)
    y1 = _tc_layer(p, g1, degp3, b1.reshape(1, D), W2.reshape(1, D))
    q = _make_sc_agg1d()(y1.reshape(N), src2, dst2, zero_n8)
    q = q[:, :N].reshape(2, N, 1)
    return _tc_final(q, y1, degp3, b2.reshape(1, 1))


# speedup vs baseline: 1.3711x; 1.1796x over previous
import dataclasses
import functools

import jax
import jax.numpy as jnp
from jax.experimental import pallas as pl
from jax.experimental.pallas import tpu as pltpu
from jax.experimental.pallas import tpu_sc as plsc

N = 10000
D = 128
E = 320000
ER = 2500
RPW = 80
LAST = ER - 31 * RPW


def _sc_compiler_params():
    cp = pltpu.CompilerParams()
    if "needs_layout_passes" in pltpu.CompilerParams.__dataclass_fields__:
        cp = dataclasses.replace(cp, needs_layout_passes=False)
    return cp


@functools.cache
def _mesh():
    return plsc.VectorSubcoreMesh(core_axis_name="c", subcore_axis_name="s")


def _stage_rows(hbm, vmem, w):
    @pl.when(w < 31)
    def _():
        off = pl.multiple_of(w * RPW, 8)
        pltpu.sync_copy(hbm.at[pl.ds(off, RPW)], vmem)

    @pl.when(w == 31)
    def _():
        pltpu.sync_copy(hbm.at[pl.ds(31 * RPW, LAST)], vmem.at[pl.ds(0, LAST)])


@functools.cache
def _make_sc_degree():
    @functools.partial(
        pl.kernel,
        out_type=jax.ShapeDtypeStruct((2, N), jnp.float32),
        mesh=_mesh(),
        scratch_types=[
            pltpu.VMEM((RPW, 128), jnp.int32),
            pltpu.VMEM((128,), jnp.float32),
            pltpu.VMEM_SHARED((N,), jnp.float32),
            pltpu.SemaphoreType.DMA,
        ],
    )
    def _sc_degree(dst_hbm, ones_hbm, zero_hbm, out_hbm,
                   dst_v, ones_v, acc_sp, sem):
        core = jax.lax.axis_index("c")
        tile = jax.lax.axis_index("s")
        w = core * 16 + tile
        nch = jnp.where(w == 31, LAST, RPW)
        _stage_rows(dst_hbm, dst_v, w)
        pltpu.sync_copy(ones_hbm, ones_v)

        @pl.when(tile == 0)
        def _():
            pltpu.sync_copy(zero_hbm, acc_sp)

        plsc.subcore_barrier()

        @pl.loop(0, nch, step=4)
        def _(j):
            for k in range(4):
                pltpu.async_copy(ones_v, acc_sp.at[dst_v.at[j + k]], sem,
                                 add=True)
            for k in range(4):
                pltpu.make_async_copy(ones_v, acc_sp.at[dst_v.at[j + k]],
                                      sem).wait()

        plsc.subcore_barrier()

        @pl.when(tile == 0)
        def _():
            pltpu.sync_copy(acc_sp, out_hbm.at[core])

    return _sc_degree


@functools.cache
def _make_sc_agg():
    @functools.partial(
        pl.kernel,
        out_type=jax.ShapeDtypeStruct((2, N, D), jnp.float32),
        mesh=_mesh(),
        scratch_types=[
            pltpu.VMEM((RPW, 128), jnp.int32),
            pltpu.VMEM((40, 128), jnp.int32),
            pltpu.VMEM((128, D), jnp.float32),
            pltpu.VMEM((128, D), jnp.float32),
            pltpu.VMEM_SHARED((N, D), jnp.float32),
            pltpu.SemaphoreType.DMA,
            pltpu.SemaphoreType.DMA,
        ],
    )
    def _agg(g_hbm, src_hbm, dst_hbm, out_hbm,
             src_v, dst_v, rows0, rows1, acc_sp, sem0, sem1):
        core = jax.lax.axis_index("c")
        tile = jax.lax.axis_index("s")
        w = core * 16 + tile
        nch = jnp.where(w == 31, LAST, RPW)
        _stage_rows(src_hbm, src_v, w)
        base = pl.multiple_of(tile * 632, 8)

        @pl.loop(0, 128)
        def _(r):
            for c in range(8):
                rows0[r, pl.ds(c * 16, 16)] = jnp.zeros((16,), jnp.float32)

        @pl.when(tile < 15)
        def _():
            for k in range(5):
                pltpu.sync_copy(rows0.at[pl.ds(0, 120)],
                                acc_sp.at[pl.ds(base + k * 120, 120)])
            pltpu.sync_copy(rows0.at[pl.ds(0, 32)],
                            acc_sp.at[pl.ds(base + 600, 32)])

        @pl.when(tile == 15)
        def _():
            for k in range(4):
                pltpu.sync_copy(rows0.at[pl.ds(0, 120)],
                                acc_sp.at[pl.ds(15 * 632 + k * 120, 120)])
            pltpu.sync_copy(rows0.at[pl.ds(0, 40)],
                            acc_sp.at[pl.ds(15 * 632 + 480, 40)])

        pltpu.async_copy(g_hbm.at[src_v.at[0]], rows0, sem0)
        plsc.subcore_barrier()

        @pl.loop(0, nch, step=2)
        def _(j):
            q = j // 40

            @pl.when(j == q * 40)
            def _():
                @pl.when(w < 31)
                def _():
                    off = pl.multiple_of(w * RPW + q * 40, 8)
                    pltpu.sync_copy(dst_hbm.at[pl.ds(off, 40)], dst_v)

                @pl.when(w == 31)
                def _():
                    pltpu.sync_copy(dst_hbm.at[pl.ds(31 * RPW, LAST)],
                                    dst_v.at[pl.ds(0, LAST)])

            jd = j - q * 40
            pltpu.async_copy(g_hbm.at[src_v.at[j + 1]], rows1, sem1)
            pltpu.make_async_copy(g_hbm.at[src_v.at[j]], rows0, sem0).wait()
            pltpu.sync_copy(rows0, acc_sp.at[dst_v.at[jd]], add=True)

            @pl.when(j + 2 < nch)
            def _():
                pltpu.async_copy(g_hbm.at[src_v.at[j + 2]], rows0, sem0)

            pltpu.make_async_copy(g_hbm.at[src_v.at[j + 1]], rows1, sem1).wait()
            pltpu.sync_copy(rows1, acc_sp.at[dst_v.at[jd + 1]], add=True)

        plsc.subcore_barrier()

        @pl.when(tile < 15)
        def _():
            pltpu.sync_copy(acc_sp.at[pl.ds(base, 632)],
                            out_hbm.at[core, pl.ds(base, 632)])

        @pl.when(tile == 15)
        def _():
            pltpu.sync_copy(acc_sp.at[pl.ds(15 * 632, N - 15 * 632)],
                            out_hbm.at[core, pl.ds(15 * 632, N - 15 * 632)])

    return _agg


@functools.cache
def _make_sc_agg1d():
    @functools.partial(
        pl.kernel,
        out_type=jax.ShapeDtypeStruct((2, N), jnp.float32),
        mesh=_mesh(),
        scratch_types=[
            pltpu.VMEM((RPW, 128), jnp.int32),
            pltpu.VMEM((RPW, 128), jnp.int32),
            pltpu.VMEM((N,), jnp.float32),
            pltpu.VMEM((RPW, 128), jnp.float32),
            pltpu.VMEM_SHARED((N,), jnp.float32),
            pltpu.SemaphoreType.DMA,
        ],
        compiler_params=_sc_compiler_params(),
    )
    def _agg1d(y_hbm, src_hbm, dst_hbm, zero_hbm, out_hbm,
               src_v, dst_v, y_l, vals_v, acc_sp, sem):
        core = jax.lax.axis_index("c")
        tile = jax.lax.axis_index("s")
        w = core * 16 + tile
        nch = jnp.where(w == 31, LAST, RPW)
        _stage_rows(src_hbm, src_v, w)
        _stage_rows(dst_hbm, dst_v, w)
        pltpu.sync_copy(y_hbm, y_l)

        @pl.when(tile == 0)
        def _():
            pltpu.sync_copy(zero_hbm, acc_sp)

        @pl.loop(0, nch)
        def _(j):
            for k in range(8):
                idx = src_v[j, pl.ds(k * 16, 16)]
                vals_v[j, pl.ds(k * 16, 16)] = plsc.load_gather(y_l, [idx])

        plsc.subcore_barrier()

        @pl.loop(0, nch, step=4)
        def _(j):
            for k in range(4):
                pltpu.async_copy(vals_v.at[j + k], acc_sp.at[dst_v.at[j + k]],
                                 sem, add=True)
            for k in range(4):
                pltpu.make_async_copy(vals_v.at[j + k],
                                      acc_sp.at[dst_v.at[j + k]], sem).wait()

        plsc.subcore_barrier()

        @pl.when(tile == 0)
        def _():
            pltpu.sync_copy(acc_sp, out_hbm.at[core])

    return _agg1d


def _tc_matmul_scale(x, W1, degp):
    def body(x_ref, w_ref, degp_ref, o_ref):
        dinv_r = jax.lax.rsqrt(degp_ref[0] + degp_ref[1] + 1.0)
        dinv = jnp.reshape(dinv_r, (N, 1))
        h = jnp.dot(x_ref[...], w_ref[...], preferred_element_type=jnp.float32)
        o_ref[...] = h * dinv

    return pl.pallas_call(
        body, out_shape=jax.ShapeDtypeStruct((N, D), jnp.float32))(x, W1, degp)


def _tc_layer(p, g1, degp, b1, W2):
    def body(p_ref, g_ref, degp_ref, b1_ref, w2_ref, o_ref):
        dinv_r = jax.lax.rsqrt(degp_ref[0] + degp_ref[1] + 1.0)
        dinv = jnp.reshape(dinv_r, (N, 1))
        acc = p_ref[0] + p_ref[1] + g_ref[...]
        h1 = jnp.maximum(acc * dinv + b1_ref[...], 0.0)
        y = jnp.dot(h1, w2_ref[...], preferred_element_type=jnp.float32) * dinv
        o_ref[...] = jnp.reshape(y, (N,))

    return pl.pallas_call(
        body, out_shape=jax.ShapeDtypeStruct((N,), jnp.float32))(
            p, g1, degp, b1, W2)


def _tc_final(q, y1, degp, b2):
    def body(q_ref, y_ref, degp_ref, b2_ref, o_ref):
        dinv = jax.lax.rsqrt(degp_ref[0] + degp_ref[1] + 1.0)
        tot = q_ref[0] + q_ref[1] + y_ref[...]
        res = jax.nn.sigmoid(tot * dinv + b2_ref[...])
        o_ref[...] = jnp.reshape(res, (N, 1))

    return pl.pallas_call(
        body, out_shape=jax.ShapeDtypeStruct((N, 1), jnp.float32))(
            q, y1, degp, b2)


def kernel(x, edge_index, W1, b1, W2, b2):
    ei = edge_index.astype(jnp.int32)
    src = ei[0].reshape(ER, 128)
    dst = ei[1].reshape(ER, 128)
    ones_n = jnp.ones((128,), jnp.float32)
    zero_n = jnp.zeros((N,), jnp.float32)

    degp = _make_sc_degree()(dst, ones_n, zero_n)
    g1 = _tc_matmul_scale(x, W1, degp)
    p = _make_sc_agg()(g1, src, dst)
    y1 = _tc_layer(p, g1, degp, b1, W2)
    q = _make_sc_agg1d()(y1, src, dst, zero_n)
    return _tc_final(q, y1, degp, b2)
